# baseline (speedup 1.0000x reference)
import functools

import jax
import jax.numpy as jnp
from jax.experimental import pallas as pl


def _fwd_kernel(x_ref, w_ref, o_ref):
    acc = jax.lax.dot_general(
        x_ref[...],
        w_ref[...],
        dimension_numbers=(((1,), (1,)), ((), ())),
        preferred_element_type=jnp.float32,
    )
    o_ref[...] = jax.nn.sigmoid(acc)


@jax.jit
def kernel(x, W):
    B, I = x.shape
    O = W.shape[0]
    BLK = 2048
    return pl.pallas_call(
        _fwd_kernel,
        grid=(B // BLK,),
        in_specs=[
            pl.BlockSpec((BLK, I), lambda i: (i, 0)),
            pl.BlockSpec((O, I), lambda i: (0, 0)),
        ],
        out_specs=pl.BlockSpec((BLK, O), lambda i: (i, 0)),
        out_shape=jax.ShapeDtypeStruct((B, O), jnp.float32),
    )(x, W)
